# baseline (speedup 1.0000x reference)
import jax
import jax.numpy as jnp
import numpy as np
from jax import lax
from jax.experimental import pallas as pl
from jax.experimental.pallas import tpu as pltpu
from jax.experimental.pallas import tpu_sc as plsc

VOCAB = 100000
HIDDEN = 128
BBOX_VOCAB = 1004
BBOX_DIM = 64
B, S = 4096, 50
N = B * S

NC, NS, L = 2, 16, 16
NW = NC * NS
TOK_PER_W = N // NW
CHUNK = 80
NCHUNK = TOK_PER_W // CHUNK
ZROW = 6 * BBOX_VOCAB
NG = CHUNK * 6 // L


def _body(tok_hbm, boxf_hbm, cst_hbm, ttab_hbm, btab_hbm, out_hbm,
          tidx_v, bidx_v, bidx2_v, trows_v, brows_v, cst_v,
          isem, gsem, osem):
    wid = lax.axis_index("s") * NC + lax.axis_index("c")
    wbase = wid * TOK_PER_W
    lanes = lax.broadcasted_iota(jnp.int32, (L,), 0)

    def stage_idx(c, tb, bb):
        base = wbase + c * CHUNK
        pltpu.async_copy(tok_hbm.at[pl.ds(base, CHUNK)], tidx_v.at[tb],
                         isem.at[bb])
        pltpu.async_copy(boxf_hbm.at[pl.ds(base * 6, CHUNK * 6)],
                         bidx_v.at[bb], isem.at[bb])

    def stage(c, tb, bb, first=False):
        base = wbase + c * CHUNK
        if not first:
            pltpu.make_async_copy(
                trows_v.at[tb], out_hbm.at[pl.ds(base, CHUNK)],
                osem.at[tb]).wait()
        pltpu.make_async_copy(tok_hbm.at[pl.ds(base, CHUNK)], tidx_v.at[tb],
                              isem.at[bb]).wait()
        pltpu.make_async_copy(boxf_hbm.at[pl.ds(base * 6, CHUNK * 6)],
                              bidx_v.at[bb], isem.at[bb]).wait()
        for g in range(NG):
            e0 = g * L
            k = e0 % 48
            raw = bidx_v[bb, pl.ds(e0, L)]
            offs = cst_v[pl.ds(k, L)]
            rem = cst_v[pl.ds(48 + k, L)]
            box0 = plsc.load_gather(bidx_v.at[bb], [(e0 + lanes) - rem])
            keep = box0 < 1001
            bidx2_v[bb, g // 5, pl.ds((g % 5) * L, L)] = (
                jnp.where(keep, raw + offs, ZROW))
        pltpu.async_copy(ttab_hbm.at[tidx_v.at[tb]], trows_v.at[tb],
                         gsem.at[bb])
        for i in range(6):
            pltpu.async_copy(btab_hbm.at[bidx2_v.at[bb, i]],
                             brows_v.at[bb, pl.ds(i * CHUNK, CHUNK)],
                             gsem.at[bb])

    def finish(c, tb, bb):
        base = wbase + c * CHUNK
        pltpu.make_async_copy(ttab_hbm.at[tidx_v.at[tb]], trows_v.at[tb],
                              gsem.at[bb]).wait()
        for i in range(6):
            pltpu.make_async_copy(btab_hbm.at[bidx2_v.at[bb, i]],
                                  brows_v.at[bb, pl.ds(i * CHUNK, CHUNK)],
                                  gsem.at[bb]).wait()

        @pl.loop(0, CHUNK)
        def _tok(t):
            e = t * 6
            for j in range(BBOX_DIM // L):
                acc = trows_v[tb, t, pl.ds(BBOX_DIM + j * L, L)]
                for i in range(6):
                    acc = acc + brows_v[bb, e + i, pl.ds(j * L, L)]
                trows_v[tb, t, pl.ds(BBOX_DIM + j * L, L)] = acc

        pltpu.async_copy(trows_v.at[tb], out_hbm.at[pl.ds(base, CHUNK)],
                         osem.at[tb])

    pltpu.sync_copy(cst_hbm, cst_v)

    stage_idx(0, 0, 0)
    stage_idx(1, 1, 1)
    stage(0, 0, 0, first=True)
    stage_idx(2, 2, 0)
    stage(1, 1, 1, first=True)
    stage_idx(3, 3, 1)
    finish(0, 0, 0)
    stage(2, 2, 0, first=True)
    finish(1, 1, 1)
    stage(3, 3, 1, first=True)

    @pl.loop(1, NCHUNK // 4)
    def _grp(k):
        c = 4 * k
        finish(c - 2, 2, 0)
        stage_idx(c, 0, 0)
        stage_idx(c + 1, 1, 1)
        finish(c - 1, 3, 1)
        stage(c, 0, 0)
        stage_idx(c + 2, 2, 0)
        finish(c, 0, 0)
        stage(c + 1, 1, 1)
        stage_idx(c + 3, 3, 1)
        finish(c + 1, 1, 1)
        stage(c + 2, 2, 0)
        stage(c + 3, 3, 1)

    finish(NCHUNK - 2, 2, 0)
    finish(NCHUNK - 1, 3, 1)
    for tb in range(4):
        pltpu.make_async_copy(trows_v.at[tb], out_hbm.at[pl.ds(wbase, CHUNK)],
                              osem.at[tb]).wait()


@jax.jit
def _run(tok_flat, boxes_flat, csts, token_table, btab):
    kern = pl.kernel(
        _body,
        out_type=jax.ShapeDtypeStruct((N, HIDDEN), jnp.float32),
        mesh=plsc.VectorSubcoreMesh(
            core_axis_name="c", subcore_axis_name="s",
            num_cores=NC, num_subcores=NS),
        scratch_types=[
            pltpu.VMEM((4, CHUNK), jnp.int32),
            pltpu.VMEM((2, CHUNK * 6), jnp.int32),
            pltpu.VMEM((2, 6, CHUNK), jnp.int32),
            pltpu.VMEM((4, CHUNK, HIDDEN), jnp.float32),
            pltpu.VMEM((2, CHUNK * 6, BBOX_DIM), jnp.float32),
            pltpu.VMEM((96,), jnp.int32),
            pltpu.SemaphoreType.DMA((2,)),
            pltpu.SemaphoreType.DMA((2,)),
            pltpu.SemaphoreType.DMA((4,)),
        ],
        compiler_params=pltpu.CompilerParams(
            use_tc_tiling_on_sc=False, needs_layout_passes=False),
    )
    return kern(tok_flat, boxes_flat, csts, token_table, btab)


_CSTS = np.concatenate([
    (np.arange(48, dtype=np.int32) % 6) * BBOX_VOCAB,
    np.arange(48, dtype=np.int32) % 6,
])


def kernel(input_tokens, input_boxes, embed_boxes, token_table, bbox_tables):
    tok_flat = input_tokens.astype(jnp.int32).reshape(N)
    boxes_flat = input_boxes.astype(jnp.int32).reshape(N * 6)
    btab = jnp.concatenate(
        [bbox_tables.reshape(6 * BBOX_VOCAB, BBOX_DIM),
         jnp.zeros((8, BBOX_DIM), jnp.float32)])
    out = _run(tok_flat, boxes_flat, jnp.asarray(_CSTS), token_table, btab)
    return out.reshape(B, S, HIDDEN)


# speedup vs baseline: 1.2246x; 1.2246x over previous
import jax
import jax.numpy as jnp
import numpy as np
from jax import lax
from jax.experimental import pallas as pl
from jax.experimental.pallas import tpu as pltpu
from jax.experimental.pallas import tpu_sc as plsc

VOCAB = 100000
HIDDEN = 128
BBOX_VOCAB = 1004
BBOX_DIM = 64
B, S = 4096, 50
N = B * S

NC, NS, L = 2, 16, 16
NW = NC * NS
TOK_PER_W = N // NW
CHUNK = 100
BPC = CHUNK // S
NCHUNK = TOK_PER_W // CHUNK
ZROW = 6 * BBOX_VOCAB
FLAT = CHUNK * 6
FLATP = 608
NG = FLATP // L
GL = 120


def _body(tok_hbm, boxf_hbm, cst_hbm, ttab_hbm, btab_hbm, out_hbm,
          tidx_v, bidx_v, bidx2_v, trows_v, brows_v, cst_v,
          isem, gsem, osem):
    wid = lax.axis_index("s") * NC + lax.axis_index("c")
    wbase = wid * TOK_PER_W
    wb0 = wid * (TOK_PER_W // S)
    lanes = lax.broadcasted_iota(jnp.int32, (L,), 0)

    def stage_idx(c, j, tb, bb):
        pad = 4 * (j % 2)
        base = pl.multiple_of(wbase + c * CHUNK - pad, 8)
        pltpu.async_copy(tok_hbm.at[pl.ds(base, CHUNK + 4)], tidx_v.at[tb],
                         isem.at[bb])
        pltpu.async_copy(boxf_hbm.at[pl.ds((base + pad) * 6, FLAT)],
                         bidx_v.at[bb, pl.ds(0, FLAT)], isem.at[bb])

    def stage(c, j, tb, bb, first=False):
        pad = 4 * (j % 2)
        base = pl.multiple_of(wbase + c * CHUNK - pad, 8)
        b0 = wb0 + c * BPC
        if not first:
            for r in range(BPC):
                pltpu.make_async_copy(
                    trows_v.at[tb, pl.ds(pad + r * S, S)], out_hbm.at[b0 + r],
                    osem.at[tb]).wait()
        pltpu.make_async_copy(tok_hbm.at[pl.ds(base, CHUNK + 4)],
                              tidx_v.at[tb], isem.at[bb]).wait()
        pltpu.make_async_copy(boxf_hbm.at[pl.ds((base + pad) * 6, FLAT)],
                              bidx_v.at[bb, pl.ds(0, FLAT)],
                              isem.at[bb]).wait()
        for g in range(NG):
            e0 = g * L
            k = e0 % 48
            raw = bidx_v[bb, pl.ds(e0, L)]
            offs = cst_v[pl.ds(k, L)]
            rem = cst_v[pl.ds(48 + k, L)]
            box0 = plsc.load_gather(bidx_v.at[bb], [(e0 + lanes) - rem])
            keep = box0 < 1001
            bidx2_v[bb, pl.ds(e0, L)] = jnp.where(keep, raw + offs, ZROW)
        pltpu.async_copy(ttab_hbm.at[tidx_v.at[tb]], trows_v.at[tb],
                         gsem.at[bb])
        for i in range(FLAT // GL):
            pltpu.async_copy(btab_hbm.at[bidx2_v.at[bb, pl.ds(i * GL, GL)]],
                             brows_v.at[bb, pl.ds(i * GL, GL)],
                             gsem.at[bb])

    def finish(c, j, tb, bb):
        pad = 4 * (j % 2)
        b0 = wb0 + c * BPC
        pltpu.make_async_copy(ttab_hbm.at[tidx_v.at[tb]], trows_v.at[tb],
                              gsem.at[bb]).wait()
        for i in range(FLAT // GL):
            pltpu.make_async_copy(
                btab_hbm.at[bidx2_v.at[bb, pl.ds(i * GL, GL)]],
                brows_v.at[bb, pl.ds(i * GL, GL)], gsem.at[bb]).wait()

        @pl.loop(0, CHUNK, unroll=2)
        def _tok(t):
            e = t * 6
            for q in range(2):
                sl = pl.ds(q * 32, 32)
                acc = brows_v[bb, e, sl]
                for i in range(1, 6):
                    acc = acc + brows_v[bb, e + i, sl]
                a, bpart = plsc.unpack(acc, format=plsc.PackFormat.INTERLEAVED)
                c0 = BBOX_DIM + q * 32
                tp = t + pad
                trows_v[tb, tp, pl.ds(c0, L)] = (
                    trows_v[tb, tp, pl.ds(c0, L)] + a)
                trows_v[tb, tp, pl.ds(c0 + L, L)] = (
                    trows_v[tb, tp, pl.ds(c0 + L, L)] + bpart)

        for r in range(BPC):
            pltpu.async_copy(trows_v.at[tb, pl.ds(pad + r * S, S)],
                             out_hbm.at[b0 + r], osem.at[tb])

    pltpu.sync_copy(cst_hbm, cst_v)

    stage_idx(0, 0, 0, 0)
    stage_idx(1, 1, 1, 1)
    stage(0, 0, 0, 0, first=True)
    stage_idx(2, 2, 2, 0)
    stage(1, 1, 1, 1, first=True)
    stage_idx(3, 3, 3, 1)
    finish(0, 0, 0, 0)
    stage(2, 2, 2, 0, first=True)
    finish(1, 1, 1, 1)
    stage(3, 3, 3, 1, first=True)

    @pl.loop(1, NCHUNK // 4)
    def _grp(k):
        c = 4 * k
        finish(c - 2, 2, 2, 0)
        stage_idx(c, 0, 0, 0)
        stage_idx(c + 1, 1, 1, 1)
        finish(c - 1, 3, 3, 1)
        stage(c, 0, 0, 0)
        stage_idx(c + 2, 2, 2, 0)
        finish(c, 0, 0, 0)
        stage(c + 1, 1, 1, 1)
        stage_idx(c + 3, 3, 3, 1)
        finish(c + 1, 1, 1, 1)
        stage(c + 2, 2, 2, 0)
        stage(c + 3, 3, 3, 1)

    finish(NCHUNK - 2, 2, 2, 0)
    finish(NCHUNK - 1, 3, 3, 1)
    for tb in range(4):
        for r in range(BPC):
            pltpu.make_async_copy(trows_v.at[tb, pl.ds(r * S, S)],
                                  out_hbm.at[wb0 + r], osem.at[tb]).wait()


@jax.jit
def _run(tok_flat, boxes_flat, csts, token_table, btab):
    kern = pl.kernel(
        _body,
        out_type=jax.ShapeDtypeStruct((B, S, HIDDEN), jnp.float32),
        mesh=plsc.VectorSubcoreMesh(
            core_axis_name="c", subcore_axis_name="s",
            num_cores=NC, num_subcores=NS),
        scratch_types=[
            pltpu.VMEM((4, CHUNK + 4), jnp.int32),
            pltpu.VMEM((2, FLATP), jnp.int32),
            pltpu.VMEM((2, FLATP), jnp.int32),
            pltpu.VMEM((4, CHUNK + 4, HIDDEN), jnp.float32),
            pltpu.VMEM((2, FLAT, BBOX_DIM), jnp.bfloat16),
            pltpu.VMEM((96,), jnp.int32),
            pltpu.SemaphoreType.DMA((2,)),
            pltpu.SemaphoreType.DMA((2,)),
            pltpu.SemaphoreType.DMA((4,)),
        ],
        compiler_params=pltpu.CompilerParams(
            use_tc_tiling_on_sc=False, needs_layout_passes=False),
    )
    return kern(tok_flat, boxes_flat, csts, token_table, btab)


_CSTS = np.concatenate([
    (np.arange(48, dtype=np.int32) % 6) * BBOX_VOCAB,
    np.arange(48, dtype=np.int32) % 6,
])


def kernel(input_tokens, input_boxes, embed_boxes, token_table, bbox_tables):
    tok_flat = input_tokens.astype(jnp.int32).reshape(N)
    boxes_flat = input_boxes.astype(jnp.int32).reshape(N * 6)
    btab = jnp.concatenate(
        [bbox_tables.reshape(6 * BBOX_VOCAB, BBOX_DIM),
         jnp.zeros((8, BBOX_DIM), jnp.float32)])
    btab = (btab.astype(jnp.bfloat16)
            .reshape(-1, 2, 2, L).transpose(0, 1, 3, 2).reshape(-1, BBOX_DIM))
    return _run(tok_flat, boxes_flat, jnp.asarray(_CSTS), token_table, btab)


# speedup vs baseline: 1.4459x; 1.1807x over previous
import jax
import jax.numpy as jnp
import numpy as np
from jax import lax
from jax.experimental import pallas as pl
from jax.experimental.pallas import tpu as pltpu
from jax.experimental.pallas import tpu_sc as plsc

VOCAB = 100000
HIDDEN = 128
BBOX_VOCAB = 1004
BBOX_DIM = 64
B, S = 4096, 50
N = B * S

NC, NS, L = 2, 16, 16
NW = NC * NS
TOK_PER_W = N // NW
CHUNK = 100
BPC = CHUNK // S
NCHUNK = TOK_PER_W // CHUNK
ZROW = 6 * BBOX_VOCAB
FLAT = CHUNK * 6
FLATP = 608
NG = FLATP // L
GL = 120


def _body(tok_hbm, boxf_hbm, cst_hbm, ttab_hbm, btab_hbm, out_hbm,
          tidx_v, bidx_v, bidx2_v, trows_v, brows_v, cst_v,
          isem, gsem, osem):
    wid = lax.axis_index("s") * NC + lax.axis_index("c")
    wbase = wid * TOK_PER_W
    wb0 = wid * (TOK_PER_W // S)
    lanes = lax.broadcasted_iota(jnp.int32, (L,), 0)

    def stage_idx(c, j, tb, bb):
        pad = 4 * (j % 2)
        base = pl.multiple_of(wbase + c * CHUNK - pad, 8)
        pltpu.async_copy(tok_hbm.at[pl.ds(base, CHUNK + 4)], tidx_v.at[tb],
                         isem.at[bb])
        pltpu.async_copy(boxf_hbm.at[pl.ds((base + pad) * 6, FLAT)],
                         bidx_v.at[bb, pl.ds(0, FLAT)], isem.at[bb])

    def stage(c, j, tb, bb, first=False):
        pad = 4 * (j % 2)
        base = pl.multiple_of(wbase + c * CHUNK - pad, 8)
        b0 = wb0 + c * BPC
        if not first:
            for r in range(BPC):
                pltpu.make_async_copy(
                    trows_v.at[tb, pl.ds(pad + r * S, S)],
                    out_hbm.at[b0 + r, pl.ds(0, S)], osem.at[tb]).wait()
        pltpu.make_async_copy(tok_hbm.at[pl.ds(base, CHUNK + 4)],
                              tidx_v.at[tb], isem.at[bb]).wait()
        pltpu.make_async_copy(boxf_hbm.at[pl.ds((base + pad) * 6, FLAT)],
                              bidx_v.at[bb, pl.ds(0, FLAT)],
                              isem.at[bb]).wait()
        for g in range(NG):
            e0 = g * L
            k = e0 % 48
            raw = bidx_v[bb, pl.ds(e0, L)]
            offs = cst_v[pl.ds(k, L)]
            rem = cst_v[pl.ds(48 + k, L)]
            box0 = plsc.load_gather(bidx_v.at[bb], [(e0 + lanes) - rem])
            keep = box0 < 1001
            bidx2_v[bb, pl.ds(e0, L)] = jnp.where(keep, raw + offs, ZROW)
        pltpu.async_copy(ttab_hbm.at[tidx_v.at[tb]], trows_v.at[tb],
                         gsem.at[bb])
        for i in range(FLAT // GL):
            pltpu.async_copy(btab_hbm.at[bidx2_v.at[bb, pl.ds(i * GL, GL)]],
                             brows_v.at[bb, pl.ds(i * GL, GL)],
                             gsem.at[bb])

    def finish(c, j, tb, bb):
        pad = 4 * (j % 2)
        b0 = wb0 + c * BPC
        pltpu.make_async_copy(ttab_hbm.at[tidx_v.at[tb]], trows_v.at[tb],
                              gsem.at[bb]).wait()
        for i in range(FLAT // GL):
            pltpu.make_async_copy(
                btab_hbm.at[bidx2_v.at[bb, pl.ds(i * GL, GL)]],
                brows_v.at[bb, pl.ds(i * GL, GL)], gsem.at[bb]).wait()

        @pl.loop(0, CHUNK, unroll=2)
        def _tok(t):
            e = t * 6
            for q in range(2):
                sl = pl.ds(q * 32, 32)
                acc = brows_v[bb, e, sl]
                for i in range(1, 6):
                    acc = acc + brows_v[bb, e + i, sl]
                a, bpart = plsc.unpack(acc, format=plsc.PackFormat.INTERLEAVED)
                c0 = BBOX_DIM + q * 32
                tp = t + pad
                trows_v[tb, tp, pl.ds(c0, L)] = (
                    trows_v[tb, tp, pl.ds(c0, L)] + a)
                trows_v[tb, tp, pl.ds(c0 + L, L)] = (
                    trows_v[tb, tp, pl.ds(c0 + L, L)] + bpart)

        for r in range(BPC):
            pltpu.async_copy(trows_v.at[tb, pl.ds(pad + r * S, S)],
                             out_hbm.at[b0 + r, pl.ds(0, S)], osem.at[tb])

    pltpu.sync_copy(cst_hbm, cst_v)

    stage_idx(0, 0, 0, 0)
    stage_idx(1, 1, 1, 1)
    stage(0, 0, 0, 0, first=True)
    stage_idx(2, 2, 2, 0)
    stage(1, 1, 1, 1, first=True)
    stage_idx(3, 3, 3, 1)
    finish(0, 0, 0, 0)
    stage(2, 2, 2, 0, first=True)
    finish(1, 1, 1, 1)
    stage(3, 3, 3, 1, first=True)

    @pl.loop(1, NCHUNK // 4)
    def _grp(k):
        c = 4 * k
        finish(c - 2, 2, 2, 0)
        stage_idx(c, 0, 0, 0)
        stage_idx(c + 1, 1, 1, 1)
        finish(c - 1, 3, 3, 1)
        stage(c, 0, 0, 0)
        stage_idx(c + 2, 2, 2, 0)
        finish(c, 0, 0, 0)
        stage(c + 1, 1, 1, 1)
        stage_idx(c + 3, 3, 3, 1)
        finish(c + 1, 1, 1, 1)
        stage(c + 2, 2, 2, 0)
        stage(c + 3, 3, 3, 1)

    finish(NCHUNK - 2, 2, 2, 0)
    finish(NCHUNK - 1, 3, 3, 1)
    for tb in range(4):
        for r in range(BPC):
            pltpu.make_async_copy(trows_v.at[tb, pl.ds(r * S, S)],
                                  out_hbm.at[wb0 + r, pl.ds(0, S)],
                                  osem.at[tb]).wait()


@jax.jit
def _run(tok_flat, boxes_flat, csts, token_table, btab):
    kern = pl.kernel(
        _body,
        out_type=jax.ShapeDtypeStruct((B, 56, HIDDEN), jnp.float32),
        mesh=plsc.VectorSubcoreMesh(
            core_axis_name="c", subcore_axis_name="s",
            num_cores=NC, num_subcores=NS),
        scratch_types=[
            pltpu.VMEM((4, CHUNK + 4), jnp.int32),
            pltpu.VMEM((2, FLATP), jnp.int32),
            pltpu.VMEM((2, FLATP), jnp.int32),
            pltpu.VMEM((4, CHUNK + 4, HIDDEN), jnp.float32),
            pltpu.VMEM((2, FLAT, BBOX_DIM), jnp.bfloat16),
            pltpu.VMEM((96,), jnp.int32),
            pltpu.SemaphoreType.DMA((2,)),
            pltpu.SemaphoreType.DMA((2,)),
            pltpu.SemaphoreType.DMA((4,)),
        ],
        compiler_params=pltpu.CompilerParams(
            use_tc_tiling_on_sc=False, needs_layout_passes=False),
    )
    return kern(tok_flat, boxes_flat, csts, token_table, btab)


_CSTS = np.concatenate([
    (np.arange(48, dtype=np.int32) % 6) * BBOX_VOCAB,
    np.arange(48, dtype=np.int32) % 6,
])


def kernel(input_tokens, input_boxes, embed_boxes, token_table, bbox_tables):
    tok_flat = input_tokens.astype(jnp.int32).reshape(N)
    boxes_flat = input_boxes.astype(jnp.int32).reshape(N * 6)
    btab = jnp.concatenate(
        [bbox_tables.reshape(6 * BBOX_VOCAB, BBOX_DIM),
         jnp.zeros((8, BBOX_DIM), jnp.float32)])
    btab = (btab.astype(jnp.bfloat16)
            .reshape(-1, 2, 2, L).transpose(0, 1, 3, 2).reshape(-1, BBOX_DIM))
    out = _run(tok_flat, boxes_flat, jnp.asarray(_CSTS), token_table, btab)
    return out[:, :S, :]


# speedup vs baseline: 1.4486x; 1.0019x over previous
import jax
import jax.numpy as jnp
import numpy as np
from jax import lax
from jax.experimental import pallas as pl
from jax.experimental.pallas import tpu as pltpu
from jax.experimental.pallas import tpu_sc as plsc

VOCAB = 100000
HIDDEN = 128
BBOX_VOCAB = 1004
BBOX_DIM = 64
B, S = 4096, 50
N = B * S

NC, NS, L = 2, 16, 16
NW = NC * NS
TOK_PER_W = N // NW
CHUNK = 100
BPC = CHUNK // S
NCHUNK = TOK_PER_W // CHUNK
ZROW = 6 * BBOX_VOCAB
FLAT = CHUNK * 6
FLATP = 608
NG = FLATP // L
GSPLIT = ((0, 304), (304, 296))


def _body(tok_hbm, boxf_hbm, cst_hbm, ttab_hbm, btab_hbm, out_hbm,
          tidx_v, bidx_v, bidx2_v, trows_v, brows_v, cst_v,
          isem, gsem, osem):
    wid = lax.axis_index("s") * NC + lax.axis_index("c")
    wbase = wid * TOK_PER_W
    wb0 = wid * (TOK_PER_W // S)
    lanes = lax.broadcasted_iota(jnp.int32, (L,), 0)

    def stage_idx(c, j, tb, bb):
        pad = 4 * (j % 2)
        base = pl.multiple_of(wbase + c * CHUNK - pad, 8)
        pltpu.async_copy(tok_hbm.at[pl.ds(base, CHUNK + 4)], tidx_v.at[tb],
                         isem.at[bb])
        pltpu.async_copy(boxf_hbm.at[pl.ds((base + pad) * 6, FLAT)],
                         bidx_v.at[bb, pl.ds(0, FLAT)], isem.at[bb])

    def stage(c, j, tb, bb, first=False):
        pad = 4 * (j % 2)
        base = pl.multiple_of(wbase + c * CHUNK - pad, 8)
        b0 = wb0 + c * BPC
        if not first:
            for r in range(BPC):
                pltpu.make_async_copy(
                    trows_v.at[tb, pl.ds(pad + r * S, S)],
                    out_hbm.at[b0 + r, pl.ds(0, S)], osem.at[tb]).wait()
        pltpu.make_async_copy(tok_hbm.at[pl.ds(base, CHUNK + 4)],
                              tidx_v.at[tb], isem.at[bb]).wait()
        pltpu.make_async_copy(boxf_hbm.at[pl.ds((base + pad) * 6, FLAT)],
                              bidx_v.at[bb, pl.ds(0, FLAT)],
                              isem.at[bb]).wait()
        for g in range(NG):
            e0 = g * L
            k = e0 % 48
            raw = bidx_v[bb, pl.ds(e0, L)]
            offs = cst_v[pl.ds(k, L)]
            rem = cst_v[pl.ds(48 + k, L)]
            box0 = plsc.load_gather(bidx_v.at[bb], [(e0 + lanes) - rem])
            keep = box0 < 1001
            bidx2_v[bb, pl.ds(e0, L)] = jnp.where(keep, raw + offs, ZROW)
        pltpu.async_copy(ttab_hbm.at[tidx_v.at[tb]], trows_v.at[tb],
                         gsem.at[bb])
        for o, n in GSPLIT:
            pltpu.async_copy(btab_hbm.at[bidx2_v.at[bb, pl.ds(o, n)]],
                             brows_v.at[bb, pl.ds(o, n)],
                             gsem.at[bb])

    def finish(c, j, tb, bb):
        pad = 4 * (j % 2)
        b0 = wb0 + c * BPC
        pltpu.make_async_copy(ttab_hbm.at[tidx_v.at[tb]], trows_v.at[tb],
                              gsem.at[bb]).wait()
        for o, n in GSPLIT:
            pltpu.make_async_copy(
                btab_hbm.at[bidx2_v.at[bb, pl.ds(o, n)]],
                brows_v.at[bb, pl.ds(o, n)], gsem.at[bb]).wait()

        @pl.loop(0, CHUNK, unroll=2)
        def _tok(t):
            e = t * 6
            for q in range(2):
                sl = pl.ds(q * 32, 32)
                acc = brows_v[bb, e, sl]
                for i in range(1, 6):
                    acc = acc + brows_v[bb, e + i, sl]
                a, bpart = plsc.unpack(acc, format=plsc.PackFormat.INTERLEAVED)
                c0 = BBOX_DIM + q * 32
                tp = t + pad
                trows_v[tb, tp, pl.ds(c0, L)] = (
                    trows_v[tb, tp, pl.ds(c0, L)] + a)
                trows_v[tb, tp, pl.ds(c0 + L, L)] = (
                    trows_v[tb, tp, pl.ds(c0 + L, L)] + bpart)

        for r in range(BPC):
            pltpu.async_copy(trows_v.at[tb, pl.ds(pad + r * S, S)],
                             out_hbm.at[b0 + r, pl.ds(0, S)], osem.at[tb])

    pltpu.sync_copy(cst_hbm, cst_v)

    stage_idx(0, 0, 0, 0)
    stage_idx(1, 1, 1, 1)
    stage(0, 0, 0, 0, first=True)
    stage_idx(2, 2, 2, 0)
    stage(1, 1, 1, 1, first=True)
    stage_idx(3, 3, 3, 1)
    finish(0, 0, 0, 0)
    stage(2, 2, 2, 0, first=True)
    finish(1, 1, 1, 1)
    stage(3, 3, 3, 1, first=True)

    @pl.loop(1, NCHUNK // 4)
    def _grp(k):
        c = 4 * k
        finish(c - 2, 2, 2, 0)
        stage_idx(c, 0, 0, 0)
        stage_idx(c + 1, 1, 1, 1)
        finish(c - 1, 3, 3, 1)
        stage(c, 0, 0, 0)
        stage_idx(c + 2, 2, 2, 0)
        finish(c, 0, 0, 0)
        stage(c + 1, 1, 1, 1)
        stage_idx(c + 3, 3, 3, 1)
        finish(c + 1, 1, 1, 1)
        stage(c + 2, 2, 2, 0)
        stage(c + 3, 3, 3, 1)

    finish(NCHUNK - 2, 2, 2, 0)
    finish(NCHUNK - 1, 3, 3, 1)
    for tb in range(4):
        for r in range(BPC):
            pltpu.make_async_copy(trows_v.at[tb, pl.ds(r * S, S)],
                                  out_hbm.at[wb0 + r, pl.ds(0, S)],
                                  osem.at[tb]).wait()


@jax.jit
def _run(tok_flat, boxes_flat, csts, token_table, btab):
    kern = pl.kernel(
        _body,
        out_type=jax.ShapeDtypeStruct((B, 56, HIDDEN), jnp.float32),
        mesh=plsc.VectorSubcoreMesh(
            core_axis_name="c", subcore_axis_name="s",
            num_cores=NC, num_subcores=NS),
        scratch_types=[
            pltpu.VMEM((4, CHUNK + 4), jnp.int32),
            pltpu.VMEM((2, FLATP), jnp.int32),
            pltpu.VMEM((2, FLATP), jnp.int32),
            pltpu.VMEM((4, CHUNK + 4, HIDDEN), jnp.float32),
            pltpu.VMEM((2, FLAT, BBOX_DIM), jnp.bfloat16),
            pltpu.VMEM((96,), jnp.int32),
            pltpu.SemaphoreType.DMA((2,)),
            pltpu.SemaphoreType.DMA((2,)),
            pltpu.SemaphoreType.DMA((4,)),
        ],
        compiler_params=pltpu.CompilerParams(
            use_tc_tiling_on_sc=False, needs_layout_passes=False),
    )
    return kern(tok_flat, boxes_flat, csts, token_table, btab)


_CSTS = np.concatenate([
    (np.arange(48, dtype=np.int32) % 6) * BBOX_VOCAB,
    np.arange(48, dtype=np.int32) % 6,
])


def kernel(input_tokens, input_boxes, embed_boxes, token_table, bbox_tables):
    tok_flat = input_tokens.astype(jnp.int32).reshape(N)
    boxes_flat = input_boxes.astype(jnp.int32).reshape(N * 6)
    btab = jnp.concatenate(
        [bbox_tables.reshape(6 * BBOX_VOCAB, BBOX_DIM),
         jnp.zeros((8, BBOX_DIM), jnp.float32)])
    btab = (btab.astype(jnp.bfloat16)
            .reshape(-1, 2, 2, L).transpose(0, 1, 3, 2).reshape(-1, BBOX_DIM))
    out = _run(tok_flat, boxes_flat, jnp.asarray(_CSTS), token_table, btab)
    return out[:, :S, :]
